# baseline (speedup 1.0000x reference)
import functools

import jax
import jax.numpy as jnp
from jax import lax
from jax.experimental import pallas as pl
from jax.experimental.pallas import tpu as pltpu
from jax.experimental.pallas import tpu_sc as plsc

VOCAB = 100000
N_EMBD = 64
BATCH = 16384

NUM_CORES = 2
NUM_SUBCORES = 16
NUM_WORKERS = NUM_CORES * NUM_SUBCORES
B_PER_W = BATCH // NUM_WORKERS
HALF = B_PER_W // 2
N_GROUPS_HALF = HALF // 16

_mesh = plsc.VectorSubcoreMesh(core_axis_name="c", subcore_axis_name="s")


@functools.partial(
    pl.kernel,
    mesh=_mesh,
    out_type=jax.ShapeDtypeStruct((BATCH, N_EMBD), jnp.float32),
    scratch_types=[
        pltpu.VMEM((B_PER_W,), jnp.int32),
        pltpu.VMEM((B_PER_W, N_EMBD), jnp.float32),
        pltpu.SemaphoreType.DMA,
        pltpu.SemaphoreType.DMA,
        pltpu.SemaphoreType.DMA,
    ],
)
def _gather(table_hbm, idx_hbm, out_hbm, idx_v, rows_v, sem_a, sem_b, sem_w):
    wid = lax.axis_index("s") * NUM_CORES + lax.axis_index("c")
    base = wid * B_PER_W

    def enqueue_half(h_base, sem):
        def body(g, _):
            v16 = idx_v[pl.ds(h_base + g * 16, 16)]
            for l in range(16):
                r = v16[l]
                pltpu.async_copy(
                    table_hbm.at[pl.ds(r, 1)],
                    rows_v.at[pl.ds(h_base + g * 16 + l, 1)],
                    sem,
                )
            return ()

        lax.fori_loop(0, N_GROUPS_HALF, body, (), unroll=4)

    def drain_half(h_base, sem):
        pltpu.make_async_copy(
            table_hbm.at[pl.ds(0, HALF)],
            rows_v.at[pl.ds(h_base, HALF)],
            sem,
        ).wait()

    pltpu.sync_copy(idx_hbm.at[pl.ds(base, HALF)], idx_v.at[pl.ds(0, HALF)])
    enqueue_half(0, sem_a)
    pltpu.sync_copy(
        idx_hbm.at[pl.ds(base + HALF, HALF)], idx_v.at[pl.ds(HALF, HALF)]
    )
    enqueue_half(HALF, sem_b)
    drain_half(0, sem_a)
    wb0 = pltpu.async_copy(
        rows_v.at[pl.ds(0, HALF)], out_hbm.at[pl.ds(base, HALF)], sem_w
    )
    drain_half(HALF, sem_b)
    wb1 = pltpu.async_copy(
        rows_v.at[pl.ds(HALF, HALF)],
        out_hbm.at[pl.ds(base + HALF, HALF)],
        sem_w,
    )
    wb0.wait()
    wb1.wait()


def kernel(x, embed_weight):
    return _gather(embed_weight, x)


# speedup vs baseline: 1.0127x; 1.0127x over previous
import functools

import jax
import jax.numpy as jnp
from jax import lax
from jax.experimental import pallas as pl
from jax.experimental.pallas import tpu as pltpu
from jax.experimental.pallas import tpu_sc as plsc

VOCAB = 100000
N_EMBD = 64
BATCH = 16384

NUM_CORES = 2
NUM_SUBCORES = 16
NUM_WORKERS = NUM_CORES * NUM_SUBCORES
B_PER_W = BATCH // NUM_WORKERS
N_GROUPS = B_PER_W // 16

_mesh = plsc.VectorSubcoreMesh(core_axis_name="c", subcore_axis_name="s")


@functools.partial(
    pl.kernel,
    mesh=_mesh,
    out_type=jax.ShapeDtypeStruct((BATCH, N_EMBD), jnp.float32),
    scratch_types=[
        pltpu.VMEM((B_PER_W,), jnp.int32),
        pltpu.VMEM((B_PER_W, N_EMBD), jnp.float32),
        pltpu.SemaphoreType.DMA,
        pltpu.SemaphoreType.DMA,
        pltpu.SemaphoreType.DMA,
    ],
    compiler_params=pltpu.CompilerParams(
        skip_device_barrier=True,
        disable_bounds_checks=True,
        disable_semaphore_checks=True,
    ),
)
def _gather(table_hbm, idx_hbm, out_hbm, idx_v, rows_v, sem_a, sem_b, sem_w):
    wid = lax.axis_index("s") * NUM_CORES + lax.axis_index("c")
    base = wid * B_PER_W
    half = B_PER_W // 2
    pltpu.sync_copy(idx_hbm.at[pl.ds(base, B_PER_W)], idx_v)

    def make_body(sem):
        def body(g, _):
            v16 = idx_v[pl.ds(g * 16, 16)]
            for l in range(16):
                r = v16[l]
                pltpu.async_copy(
                    table_hbm.at[pl.ds(r, 1)],
                    rows_v.at[pl.ds(g * 16 + l, 1)],
                    sem,
                )
            return ()

        return body

    lax.fori_loop(0, N_GROUPS // 2, make_body(sem_a), (), unroll=2)
    lax.fori_loop(N_GROUPS // 2, N_GROUPS, make_body(sem_b), (), unroll=2)
    pltpu.make_async_copy(
        table_hbm.at[pl.ds(0, half)], rows_v.at[pl.ds(0, half)], sem_a
    ).wait()
    wb0 = pltpu.async_copy(
        rows_v.at[pl.ds(0, half)], out_hbm.at[pl.ds(base, half)], sem_w
    )
    pltpu.make_async_copy(
        table_hbm.at[pl.ds(0, half)], rows_v.at[pl.ds(half, half)], sem_b
    ).wait()
    wb1 = pltpu.async_copy(
        rows_v.at[pl.ds(half, half)],
        out_hbm.at[pl.ds(base + half, half)],
        sem_w,
    )
    wb0.wait()
    wb1.wait()


def kernel(x, embed_weight):
    return _gather(embed_weight, x)


# speedup vs baseline: 1.0258x; 1.0130x over previous
import functools

import jax
import jax.numpy as jnp
from jax import lax
from jax.experimental import pallas as pl
from jax.experimental.pallas import tpu as pltpu
from jax.experimental.pallas import tpu_sc as plsc

VOCAB = 100000
N_EMBD = 64
BATCH = 16384

NUM_CORES = 2
NUM_SUBCORES = 16
NUM_WORKERS = NUM_CORES * NUM_SUBCORES
B_PER_W = BATCH // NUM_WORKERS
N_GROUPS = B_PER_W // 16

_mesh = plsc.VectorSubcoreMesh(core_axis_name="c", subcore_axis_name="s")


@functools.partial(
    pl.kernel,
    mesh=_mesh,
    out_type=jax.ShapeDtypeStruct((BATCH, N_EMBD), jnp.float32),
    scratch_types=[
        pltpu.VMEM((B_PER_W,), jnp.int32),
        pltpu.VMEM((B_PER_W, N_EMBD), jnp.float32),
        pltpu.SemaphoreType.DMA,
    ],
    compiler_params=pltpu.CompilerParams(
        skip_device_barrier=True,
        disable_bounds_checks=True,
        disable_semaphore_checks=True,
    ),
)
def _gather(table_hbm, idx_hbm, out_hbm, idx_v, rows_v, sem):
    wid = lax.axis_index("s") * NUM_CORES + lax.axis_index("c")
    base = wid * B_PER_W
    pltpu.sync_copy(idx_hbm.at[pl.ds(base, B_PER_W)], idx_v)

    def body(g, _):
        v16 = idx_v[pl.ds(g * 16, 16)]
        for l in range(16):
            r = v16[l]
            pltpu.async_copy(
                table_hbm.at[pl.ds(r, 1)],
                rows_v.at[pl.ds(g * 16 + l, 1)],
                sem,
            )
        return ()

    lax.fori_loop(0, N_GROUPS, body, (), unroll=4)
    pltpu.make_async_copy(
        table_hbm.at[pl.ds(0, B_PER_W)], rows_v, sem
    ).wait()
    pltpu.sync_copy(rows_v, out_hbm.at[pl.ds(base, B_PER_W)])


def kernel(x, embed_weight):
    return _gather(embed_weight, x)


# speedup vs baseline: 1.0270x; 1.0011x over previous
import functools

import jax
import jax.numpy as jnp
from jax import lax
from jax.experimental import pallas as pl
from jax.experimental.pallas import tpu as pltpu
from jax.experimental.pallas import tpu_sc as plsc

VOCAB = 100000
N_EMBD = 64
BATCH = 16384

NUM_CORES = 2
NUM_SUBCORES = 16
NUM_WORKERS = NUM_CORES * NUM_SUBCORES
B_PER_W = BATCH // NUM_WORKERS
N_GROUPS = B_PER_W // 16

_mesh = plsc.VectorSubcoreMesh(core_axis_name="c", subcore_axis_name="s")


@functools.partial(
    pl.kernel,
    mesh=_mesh,
    out_type=jax.ShapeDtypeStruct((BATCH, N_EMBD), jnp.float32),
    scratch_types=[
        pltpu.VMEM((B_PER_W,), jnp.int32),
        pltpu.VMEM((B_PER_W, N_EMBD), jnp.float32),
        pltpu.SemaphoreType.DMA,
    ],
    compiler_params=pltpu.CompilerParams(
        skip_device_barrier=True,
        disable_bounds_checks=True,
        disable_semaphore_checks=True,
    ),
)
def _gather(table_hbm, idx_hbm, out_hbm, idx_v, rows_v, sem):
    wid = lax.axis_index("s") * NUM_CORES + lax.axis_index("c")
    base = wid * B_PER_W
    pltpu.sync_copy(idx_hbm.at[pl.ds(base, B_PER_W)], idx_v)

    @plsc.parallel_loop(0, N_GROUPS, unroll=4)
    def _body(g):
        v16 = idx_v[pl.ds(g * 16, 16)]
        for l in range(16):
            r = v16[l]
            pltpu.async_copy(
                table_hbm.at[pl.ds(r, 1)],
                rows_v.at[pl.ds(g * 16 + l, 1)],
                sem,
            )
    pltpu.make_async_copy(
        table_hbm.at[pl.ds(0, B_PER_W)], rows_v, sem
    ).wait()
    pltpu.sync_copy(rows_v, out_hbm.at[pl.ds(base, B_PER_W)])


def kernel(x, embed_weight):
    return _gather(embed_weight, x)
